# Initial kernel scaffold; baseline (speedup 1.0000x reference)
import jax
import jax.numpy as jnp
from jax.experimental import pallas as pl


def kernel(numbers, energy, table):
    raise NotImplementedError("write your pallas kernel here")


# speedup vs baseline: 1.3457x; 1.3457x over previous
import functools

import jax
import jax.numpy as jnp
from jax import lax
from jax.experimental import pallas as pl
from jax.experimental.pallas import tpu as pltpu
from jax.experimental.pallas import tpu_sc as plsc

N = 4194304
NUM_TYPES = 64
NC = 2
NS = 16
L = 16
NW = NC * NS
PER_W = N // NW
CHUNK = 16384
NCHUNK = PER_W // CHUNK


def _sc_kernel(numbers_hbm, energy_hbm, table_hbm, out_hbm, tbl_v, idx_v, en_v):
    wid = lax.axis_index("s") * NC + lax.axis_index("c")
    pltpu.sync_copy(table_hbm, tbl_v)

    def chunk_body(c, _):
        base = wid * PER_W + c * CHUNK
        pltpu.sync_copy(numbers_hbm.at[pl.ds(base, CHUNK)], idx_v)
        pltpu.sync_copy(energy_hbm.at[pl.ds(base, CHUNK)], en_v)

        def vec_body(i, _):
            s = pl.ds(i * L, L)
            idx = idx_v[s]
            sh = plsc.load_gather(tbl_v, [idx])
            en_v[s] = en_v[s] + sh
            return 0

        lax.fori_loop(0, CHUNK // L, vec_body, 0)
        pltpu.sync_copy(en_v, out_hbm.at[pl.ds(base, CHUNK)])
        return 0

    lax.fori_loop(0, NCHUNK, chunk_body, 0)


def kernel(numbers, energy, table):
    tbl_flat = table.reshape(NUM_TYPES)
    mesh = plsc.VectorSubcoreMesh(core_axis_name="c", subcore_axis_name="s")
    run = functools.partial(
        pl.kernel,
        mesh=mesh,
        out_type=jax.ShapeDtypeStruct((N,), jnp.float32),
        scratch_types=[
            pltpu.VMEM((NUM_TYPES,), jnp.float32),
            pltpu.VMEM((CHUNK,), jnp.int32),
            pltpu.VMEM((CHUNK,), jnp.float32),
        ],
        compiler_params=pltpu.CompilerParams(needs_layout_passes=False),
    )(_sc_kernel)
    return run(numbers, energy, tbl_flat)


# speedup vs baseline: 2.0298x; 1.5084x over previous
import functools

import jax
import jax.numpy as jnp
from jax import lax
from jax.experimental import pallas as pl
from jax.experimental.pallas import tpu as pltpu
from jax.experimental.pallas import tpu_sc as plsc

N = 4194304
NUM_TYPES = 64
NC = 2
NS = 16
L = 16
NW = NC * NS
PER_W = N // NW
CHUNK = 16384
NCHUNK = PER_W // CHUNK


def _sc_kernel(numbers_hbm, energy_hbm, table_hbm, out_hbm, tbl_v, idx_v, en_v):
    wid = lax.axis_index("s") * NC + lax.axis_index("c")
    pltpu.sync_copy(table_hbm, tbl_v)

    def chunk_body(c, _):
        base = wid * PER_W + c * CHUNK
        pltpu.sync_copy(numbers_hbm.at[pl.ds(base, CHUNK)], idx_v)
        pltpu.sync_copy(energy_hbm.at[pl.ds(base, CHUNK)], en_v)

        @plsc.parallel_loop(0, CHUNK // L, unroll=8)
        def _(i):
            s = pl.ds(i * L, L)
            sh = plsc.load_gather(tbl_v, [idx_v[s]])
            plsc.addupdate(en_v.at[s], sh)
        pltpu.sync_copy(en_v, out_hbm.at[pl.ds(base, CHUNK)])
        return 0

    lax.fori_loop(0, NCHUNK, chunk_body, 0)


def kernel(numbers, energy, table):
    tbl_flat = table.reshape(NUM_TYPES)
    mesh = plsc.VectorSubcoreMesh(core_axis_name="c", subcore_axis_name="s")
    run = functools.partial(
        pl.kernel,
        mesh=mesh,
        out_type=jax.ShapeDtypeStruct((N,), jnp.float32),
        scratch_types=[
            pltpu.VMEM((NUM_TYPES,), jnp.float32),
            pltpu.VMEM((CHUNK,), jnp.int32),
            pltpu.VMEM((CHUNK,), jnp.float32),
        ],
        compiler_params=pltpu.CompilerParams(needs_layout_passes=False),
    )(_sc_kernel)
    return run(numbers, energy, tbl_flat)


# speedup vs baseline: 2.9481x; 1.4524x over previous
import functools

import jax
import jax.numpy as jnp
from jax import lax
from jax.experimental import pallas as pl
from jax.experimental.pallas import tpu as pltpu
from jax.experimental.pallas import tpu_sc as plsc

N = 4194304
NUM_TYPES = 64
NC = 2
NS = 16
L = 16
NW = NC * NS
PER_W = N // NW
CHUNK = 16384
NCHUNK = PER_W // CHUNK
NBUF = 3


def _sc_kernel(numbers_hbm, energy_hbm, table_hbm, out_hbm, tbl_v, *scratch):
    idx_bufs = scratch[0:NBUF]
    en_bufs = scratch[NBUF:2 * NBUF]
    nsems = scratch[2 * NBUF:3 * NBUF]
    esems = scratch[3 * NBUF:4 * NBUF]
    osems = scratch[4 * NBUF:5 * NBUF]

    wid = lax.axis_index("s") * NC + lax.axis_index("c")
    wbase = wid * PER_W
    pltpu.sync_copy(table_hbm, tbl_v)

    in_handles = {}
    out_handles = {}

    def start_in(c):
        b = c % NBUF
        s = pl.ds(wbase + c * CHUNK, CHUNK)
        hn = pltpu.async_copy(numbers_hbm.at[s], idx_bufs[b], nsems[b])
        he = pltpu.async_copy(energy_hbm.at[s], en_bufs[b], esems[b])
        in_handles[c] = (hn, he)

    def start_out(c):
        b = c % NBUF
        s = pl.ds(wbase + c * CHUNK, CHUNK)
        out_handles[c] = pltpu.async_copy(en_bufs[b], out_hbm.at[s], osems[b])

    for c in range(NBUF - 1):
        start_in(c)

    for c in range(NCHUNK):
        b = c % NBUF
        hn, he = in_handles.pop(c)
        hn.wait()
        he.wait()

        idx_b = idx_bufs[b]
        en_b = en_bufs[b]

        @plsc.parallel_loop(0, CHUNK // L, unroll=8)
        def _(i):
            s = pl.ds(i * L, L)
            sh = plsc.load_gather(tbl_v, [idx_b[s]])
            plsc.addupdate(en_b.at[s], sh)

        start_out(c)
        n = c + NBUF - 1
        if n < NCHUNK:
            if c >= 1:
                out_handles.pop(c - 1).wait()
            start_in(n)

    for c in sorted(out_handles):
        out_handles[c].wait()


def kernel(numbers, energy, table):
    tbl_flat = table.reshape(NUM_TYPES)
    mesh = plsc.VectorSubcoreMesh(core_axis_name="c", subcore_axis_name="s")
    run = functools.partial(
        pl.kernel,
        mesh=mesh,
        out_type=jax.ShapeDtypeStruct((N,), jnp.float32),
        scratch_types=(
            [pltpu.VMEM((NUM_TYPES,), jnp.float32)]
            + [pltpu.VMEM((CHUNK,), jnp.int32) for _ in range(NBUF)]
            + [pltpu.VMEM((CHUNK,), jnp.float32) for _ in range(NBUF)]
            + [pltpu.SemaphoreType.DMA for _ in range(3 * NBUF)]
        ),
        compiler_params=pltpu.CompilerParams(needs_layout_passes=False),
    )(_sc_kernel)
    return run(numbers, energy, tbl_flat)


# speedup vs baseline: 3.0397x; 1.0311x over previous
import functools

import jax
import jax.numpy as jnp
from jax import lax
from jax.experimental import pallas as pl
from jax.experimental.pallas import tpu as pltpu
from jax.experimental.pallas import tpu_sc as plsc

N = 4194304
NUM_TYPES = 64
NC = 2
NS = 16
L = 16
NW = NC * NS
PER_W = N // NW
CHUNK = 8192
NCHUNK = PER_W // CHUNK
NBUF = 6


def _sc_kernel(numbers_hbm, energy_hbm, table_hbm, out_hbm, tbl_v, *scratch):
    idx_bufs = scratch[0:NBUF]
    en_bufs = scratch[NBUF:2 * NBUF]
    nsems = scratch[2 * NBUF:3 * NBUF]
    esems = scratch[3 * NBUF:4 * NBUF]
    osems = scratch[4 * NBUF:5 * NBUF]

    wid = lax.axis_index("s") * NC + lax.axis_index("c")
    wbase = wid * PER_W
    pltpu.sync_copy(table_hbm, tbl_v)

    in_handles = {}
    out_handles = {}

    def start_in(c):
        b = c % NBUF
        s = pl.ds(wbase + c * CHUNK, CHUNK)
        hn = pltpu.async_copy(numbers_hbm.at[s], idx_bufs[b], nsems[b])
        he = pltpu.async_copy(energy_hbm.at[s], en_bufs[b], esems[b])
        in_handles[c] = (hn, he)

    def start_out(c):
        b = c % NBUF
        s = pl.ds(wbase + c * CHUNK, CHUNK)
        out_handles[c] = pltpu.async_copy(en_bufs[b], out_hbm.at[s], osems[b])

    for c in range(NBUF - 1):
        start_in(c)

    for c in range(NCHUNK):
        b = c % NBUF
        hn, he = in_handles.pop(c)
        hn.wait()
        he.wait()

        idx_b = idx_bufs[b]
        en_b = en_bufs[b]

        @plsc.parallel_loop(0, CHUNK // L, unroll=8)
        def _(i):
            s = pl.ds(i * L, L)
            sh = plsc.load_gather(tbl_v, [idx_b[s]])
            plsc.addupdate(en_b.at[s], sh)

        start_out(c)
        n = c + NBUF - 1
        if n < NCHUNK:
            if c >= 1:
                out_handles.pop(c - 1).wait()
            start_in(n)

    for c in sorted(out_handles):
        out_handles[c].wait()


def kernel(numbers, energy, table):
    tbl_flat = table.reshape(NUM_TYPES)
    mesh = plsc.VectorSubcoreMesh(core_axis_name="c", subcore_axis_name="s")
    run = functools.partial(
        pl.kernel,
        mesh=mesh,
        out_type=jax.ShapeDtypeStruct((N,), jnp.float32),
        scratch_types=(
            [pltpu.VMEM((NUM_TYPES,), jnp.float32)]
            + [pltpu.VMEM((CHUNK,), jnp.int32) for _ in range(NBUF)]
            + [pltpu.VMEM((CHUNK,), jnp.float32) for _ in range(NBUF)]
            + [pltpu.SemaphoreType.DMA for _ in range(3 * NBUF)]
        ),
        compiler_params=pltpu.CompilerParams(needs_layout_passes=False),
    )(_sc_kernel)
    return run(numbers, energy, tbl_flat)
